# Initial kernel scaffold; baseline (speedup 1.0000x reference)
import jax
import jax.numpy as jnp
from jax.experimental import pallas as pl


def kernel(g, h, r, norm, W):
    raise NotImplementedError("write your pallas kernel here")


# speedup vs baseline: 1.8206x; 1.8206x over previous
import functools

import jax
import jax.numpy as jnp
from jax import lax
from jax.experimental import pallas as pl
from jax.experimental.pallas import tpu as pltpu
from jax.experimental.pallas import tpu_sc as plsc

N_ROWS = 100000
D = 128
NUM_CORES = 2
NUM_SUBCORES = 16
NW = NUM_CORES * NUM_SUBCORES
CHUNK = 400
NCHUNKS = N_ROWS // CHUNK

_mesh = plsc.VectorSubcoreMesh(core_axis_name="c", subcore_axis_name="s")


@functools.partial(
    pl.kernel,
    mesh=_mesh,
    out_type=jax.ShapeDtypeStruct((N_ROWS, D), jnp.float32),
    scratch_types=[
        pltpu.VMEM((CHUNK,), jnp.int32),
        pltpu.VMEM((CHUNK, D), jnp.float32),
        pltpu.SemaphoreType.DMA,
    ],
)
def _gather(table_hbm, idx_hbm, out_hbm, idx_v, rows_v, sem):
    wid = lax.axis_index("s") * NUM_CORES + lax.axis_index("c")
    nw_chunks = (NCHUNKS - wid + NW - 1) // NW

    def step(t, carry):
        off = (wid + t * NW) * CHUNK
        pltpu.sync_copy(idx_hbm.at[pl.ds(off, CHUNK)], idx_v)
        pltpu.async_copy(table_hbm.at[idx_v], rows_v, sem).wait()
        pltpu.sync_copy(rows_v, out_hbm.at[pl.ds(off, CHUNK)])
        return carry

    lax.fori_loop(0, nw_chunks, step, 0)


def kernel(g, h, r, norm, W):
    idx = h.reshape(-1).astype(jnp.int32)
    return _gather(W, idx)


# speedup vs baseline: 2.0287x; 1.1143x over previous
import functools

import jax
import jax.numpy as jnp
from jax import lax
from jax.experimental import pallas as pl
from jax.experimental.pallas import tpu as pltpu
from jax.experimental.pallas import tpu_sc as plsc

N_ROWS = 100000
D = 128
NUM_CORES = 2
NUM_SUBCORES = 16
NW = NUM_CORES * NUM_SUBCORES
CHUNK = 400
NCHUNKS = N_ROWS // CHUNK
NMAX = (NCHUNKS + NW - 1) // NW

_mesh = plsc.VectorSubcoreMesh(core_axis_name="c", subcore_axis_name="s")


@functools.partial(
    pl.kernel,
    mesh=_mesh,
    out_type=jax.ShapeDtypeStruct((N_ROWS, D), jnp.float32),
    scratch_types=[
        pltpu.VMEM((CHUNK,), jnp.int32),
        pltpu.VMEM((CHUNK,), jnp.int32),
        pltpu.VMEM((CHUNK, D), jnp.float32),
        pltpu.VMEM((CHUNK, D), jnp.float32),
        pltpu.SemaphoreType.DMA,
        pltpu.SemaphoreType.DMA,
    ],
)
def _gather(table_hbm, idx_hbm, out_hbm, idx0, idx1, rows0, rows1,
            sem0, sem1):
    wid = lax.axis_index("s") * NUM_CORES + lax.axis_index("c")
    bufs = ((idx0, rows0, sem0), (idx1, rows1, sem1))

    def start(t, buf):
        idx_v, rows_v, sem = bufs[buf]
        c = wid + t * NW

        @pl.when(c < NCHUNKS)
        def _():
            off = c * CHUNK
            pltpu.sync_copy(idx_hbm.at[pl.ds(off, CHUNK)], idx_v)
            pltpu.async_copy(table_hbm.at[idx_v], rows_v, sem)

    start(0, 0)
    for t in range(NMAX):
        buf = t % 2
        if t + 1 < NMAX:
            start(t + 1, 1 - buf)
        idx_v, rows_v, sem = bufs[buf]
        c = wid + t * NW

        @pl.when(c < NCHUNKS)
        def _():
            pltpu.make_async_copy(table_hbm.at[idx_v], rows_v, sem).wait()
            pltpu.sync_copy(rows_v, out_hbm.at[pl.ds(c * CHUNK, CHUNK)])


def kernel(g, h, r, norm, W):
    idx = h.reshape(-1).astype(jnp.int32)
    return _gather(W, idx)


# speedup vs baseline: 2.0307x; 1.0010x over previous
import functools

import jax
import jax.numpy as jnp
from jax import lax
from jax.experimental import pallas as pl
from jax.experimental.pallas import tpu as pltpu
from jax.experimental.pallas import tpu_sc as plsc

N_ROWS = 100000
D = 128
NUM_CORES = 2
NUM_SUBCORES = 16
NW = NUM_CORES * NUM_SUBCORES
CHUNK = 400
NCHUNKS = N_ROWS // CHUNK
NMAX = (NCHUNKS + NW - 1) // NW

_mesh = plsc.VectorSubcoreMesh(core_axis_name="c", subcore_axis_name="s")


@functools.partial(
    pl.kernel,
    mesh=_mesh,
    out_type=jax.ShapeDtypeStruct((N_ROWS, D), jnp.float32),
    scratch_types=[
        pltpu.VMEM((CHUNK,), jnp.int32),
        pltpu.VMEM((CHUNK,), jnp.int32),
        pltpu.VMEM((CHUNK, D), jnp.float32),
        pltpu.VMEM((CHUNK, D), jnp.float32),
        pltpu.SemaphoreType.DMA,
        pltpu.SemaphoreType.DMA,
        pltpu.SemaphoreType.DMA,
        pltpu.SemaphoreType.DMA,
    ],
)
def _gather(table_hbm, idx_hbm, out_hbm, idx0, idx1, rows0, rows1,
            gsem0, gsem1, wsem0, wsem1):
    wid = lax.axis_index("s") * NUM_CORES + lax.axis_index("c")
    bufs = ((idx0, rows0, gsem0, wsem0), (idx1, rows1, gsem1, wsem1))

    def chunk_id(t):
        return wid + t * NW

    def start_gather(t):
        idx_v, rows_v, gsem, _ = bufs[t % 2]
        c = chunk_id(t)

        @pl.when(c < NCHUNKS)
        def _():
            pltpu.sync_copy(idx_hbm.at[pl.ds(c * CHUNK, CHUNK)], idx_v)
            pltpu.async_copy(table_hbm.at[idx_v], rows_v, gsem)

    def start_write(t):
        idx_v, rows_v, gsem, wsem = bufs[t % 2]
        c = chunk_id(t)

        @pl.when(c < NCHUNKS)
        def _():
            pltpu.make_async_copy(table_hbm.at[idx_v], rows_v, gsem).wait()
            pltpu.async_copy(rows_v, out_hbm.at[pl.ds(c * CHUNK, CHUNK)], wsem)

    def wait_write(t):
        _, rows_v, _, wsem = bufs[t % 2]
        c = chunk_id(t)

        @pl.when(c < NCHUNKS)
        def _():
            pltpu.make_async_copy(
                rows_v, out_hbm.at[pl.ds(c * CHUNK, CHUNK)], wsem).wait()

    start_gather(0)
    for t in range(NMAX):
        if t + 1 < NMAX:
            if t >= 1:
                wait_write(t - 1)
            start_gather(t + 1)
        start_write(t)
    wait_write(NMAX - 2)
    wait_write(NMAX - 1)


def kernel(g, h, r, norm, W):
    idx = h.reshape(-1).astype(jnp.int32)
    return _gather(W, idx)
